# baseline (speedup 1.0000x reference)
import functools

import jax
import jax.numpy as jnp
from jax import lax
from jax.experimental import pallas as pl
from jax.experimental.pallas import tpu as pltpu
from jax.experimental.pallas import tpu_sc as plsc

NC = 2
NS = 16
NW = NC * NS
NBUF = 8
L = 16

def _emb_kernel(nh, emb, x5_hbm, table_hbm, out5_hbm, xv, gbufs, tbufs, gsems, ssems):
    wid = lax.axis_index("s") * NC + lax.axis_index("c")
    nth = nh // 8
    for th in range(nth):
        pltpu.sync_copy(x5_hbm.at[th, wid], xv.at[th])

    def gather(b, h):
        th = h // 8
        off = pl.multiple_of((h % 8) * 128, 128)
        pltpu.async_copy(
            table_hbm.at[xv.at[th, pl.ds(off, 128)]], gbufs.at[b], gsems.at[b]
        )

    def gwait(b):
        pltpu.make_async_copy(
            table_hbm.at[pl.ds(0, 128)], gbufs.at[b], gsems.at[b]
        ).wait()

    def transpose(b):
        base0 = lax.iota(jnp.int32, L) * 128
        base1 = base0 + (L * 128)

        @plsc.parallel_loop(0, 128, step=8, unroll=4)
        def _tloop(l0):
            vs = []
            for dl in range(8):
                vs.append(gbufs[b, l0 + dl, pl.ds(0, L)])
                vs.append(gbufs[b, l0 + dl, pl.ds(L, L)])
            for dl in range(8):
                plsc.store_scatter(tbufs.at[b], [base0 + (l0 + dl)], vs[2 * dl])
                plsc.store_scatter(tbufs.at[b], [base1 + (l0 + dl)], vs[2 * dl + 1])

    def scatter(b, h):
        for te in range(emb // 8):
            pltpu.async_copy(
                tbufs.at[b, pl.ds(te * 1024, 1024)], out5_hbm.at[h, te, wid], ssems.at[b]
            )

    def swait(b):
        for te in range(emb // 8):
            pltpu.make_async_copy(
                tbufs.at[b, pl.ds(te * 1024, 1024)], out5_hbm.at[0, te, wid], ssems.at[b]
            ).wait()

    for b in range(NBUF):
        gather(b, b)

    for b in range(NBUF):
        gwait(b)
        transpose(b)
        gather(b, NBUF + b)
        scatter(b, b)

    def body(g, carry):
        h0 = g * NBUF
        for b in range(NBUF):
            gwait(b)
            swait(b)
            transpose(b)
            gather(b, h0 + NBUF + b)
            scatter(b, h0 + b)
        return carry

    lax.fori_loop(1, nh // NBUF - 1, body, 0)

    h0 = nh - NBUF
    for b in range(NBUF):
        gwait(b)
        swait(b)
        transpose(b)
        scatter(b, h0 + b)
    for b in range(NBUF):
        swait(b)


def kernel(x, table):
    bsz, nh = x.shape
    v, emb = table.shape
    assert bsz % (NW * 128) == 0 and nh % 8 == 0 and emb % 8 == 0
    nth = nh // 8

    x5 = (
        x.astype(jnp.int32)
        .reshape(NW, 128, nth, 8)
        .transpose(2, 0, 3, 1)
        .reshape(nth, NW, 1024)
    )

    mesh = plsc.VectorSubcoreMesh(
        core_axis_name="c", subcore_axis_name="s", num_cores=NC, num_subcores=NS
    )
    run = pl.kernel(
        functools.partial(_emb_kernel, nh, emb),
        mesh=mesh,
        out_type=jax.ShapeDtypeStruct((nh, emb // 8, NW, 1024), jnp.float32),
        scratch_types=[
            pltpu.VMEM((nth, 1024), jnp.int32),
            pltpu.VMEM((NBUF, 128, emb), jnp.float32),
            pltpu.VMEM((NBUF, emb * 128), jnp.float32),
            pltpu.SemaphoreType.DMA((NBUF,)),
            pltpu.SemaphoreType.DMA((NBUF,)),
        ],
        compiler_params=pltpu.CompilerParams(use_tc_tiling_on_sc=False, needs_layout_passes=False, disable_bounds_checks=True),
    )
    out5 = run(x5, table)
    return (
        out5.reshape(nh, emb // 8, NW, 8, 128)
        .transpose(2, 4, 0, 1, 3)
        .reshape(bsz, nh, emb)
    )


# speedup vs baseline: 1.0258x; 1.0258x over previous
import functools

import jax
import jax.numpy as jnp
from jax import lax
from jax.experimental import pallas as pl
from jax.experimental.pallas import tpu as pltpu
from jax.experimental.pallas import tpu_sc as plsc

NC = 2
NS = 16
NW = NC * NS
NBUF = 8
L = 16

def _emb_kernel(nh, emb, x5_hbm, table_hbm, out5_hbm, xv, gbufs, tbufs, gsems, ssems):
    wid = lax.axis_index("s") * NC + lax.axis_index("c")
    nth = nh // 8
    for th in range(nth):
        pltpu.sync_copy(x5_hbm.at[th, wid], xv.at[th])

    def gather(b, h):
        th = h // 8
        off = pl.multiple_of((h % 8) * 128, 128)
        pltpu.async_copy(
            table_hbm.at[xv.at[th, pl.ds(off, 128)]], gbufs.at[b], gsems.at[b]
        )

    def gwait(b):
        pltpu.make_async_copy(
            table_hbm.at[pl.ds(0, 128)], gbufs.at[b], gsems.at[b]
        ).wait()

    def transpose(b):
        base0 = lax.iota(jnp.int32, L) * 128
        base1 = base0 + (L * 128)

        @plsc.parallel_loop(0, 128, step=4, unroll=8)
        def _tloop(l0):
            vs = []
            for dl in range(4):
                vs.append(gbufs[b, l0 + dl, pl.ds(0, L)])
                vs.append(gbufs[b, l0 + dl, pl.ds(L, L)])
            for dl in range(4):
                plsc.store_scatter(tbufs.at[b], [base0 + (l0 + dl)], vs[2 * dl])
                plsc.store_scatter(tbufs.at[b], [base1 + (l0 + dl)], vs[2 * dl + 1])

    def scatter(b, h):
        for te in range(emb // 8):
            pltpu.async_copy(
                tbufs.at[b, pl.ds(te * 1024, 1024)], out5_hbm.at[h, te, wid], ssems.at[b]
            )

    def swait(b):
        for te in range(emb // 8):
            pltpu.make_async_copy(
                tbufs.at[b, pl.ds(te * 1024, 1024)], out5_hbm.at[0, te, wid], ssems.at[b]
            ).wait()

    for b in range(NBUF):
        gather(b, b)

    for b in range(NBUF):
        gwait(b)
        transpose(b)
        gather(b, NBUF + b)
        scatter(b, b)

    def body(g, carry):
        h0 = g * NBUF
        for b in range(NBUF):
            gwait(b)
            swait(b)
            transpose(b)
            gather(b, h0 + NBUF + b)
            scatter(b, h0 + b)
        return carry

    lax.fori_loop(1, nh // NBUF - 1, body, 0)

    h0 = nh - NBUF
    for b in range(NBUF):
        gwait(b)
        swait(b)
        transpose(b)
        scatter(b, h0 + b)
    for b in range(NBUF):
        swait(b)


def kernel(x, table):
    bsz, nh = x.shape
    v, emb = table.shape
    assert bsz % (NW * 128) == 0 and nh % 8 == 0 and emb % 8 == 0
    nth = nh // 8

    x5 = (
        x.astype(jnp.int32)
        .reshape(NW, 128, nth, 8)
        .transpose(2, 0, 3, 1)
        .reshape(nth, NW, 1024)
    )

    mesh = plsc.VectorSubcoreMesh(
        core_axis_name="c", subcore_axis_name="s", num_cores=NC, num_subcores=NS
    )
    run = pl.kernel(
        functools.partial(_emb_kernel, nh, emb),
        mesh=mesh,
        out_type=jax.ShapeDtypeStruct((nh, emb // 8, NW, 1024), jnp.float32),
        scratch_types=[
            pltpu.VMEM((nth, 1024), jnp.int32),
            pltpu.VMEM((NBUF, 128, emb), jnp.float32),
            pltpu.VMEM((NBUF, emb * 128), jnp.float32),
            pltpu.SemaphoreType.DMA((NBUF,)),
            pltpu.SemaphoreType.DMA((NBUF,)),
        ],
        compiler_params=pltpu.CompilerParams(use_tc_tiling_on_sc=False, needs_layout_passes=False, disable_bounds_checks=True),
    )
    out5 = run(x5, table)
    return (
        out5.reshape(nh, emb // 8, NW, 8, 128)
        .transpose(2, 4, 0, 1, 3)
        .reshape(bsz, nh, emb)
    )
